# baseline (speedup 1.0000x reference)
import functools
import jax
import jax.numpy as jnp
from jax import lax
from jax.experimental import pallas as pl
from jax.experimental.pallas import tpu as pltpu
from jax.experimental.pallas import tpu_sc as plsc

N_NODES = 10000
N_EDGES = 160000
HEAD = 8
CHANNEL = 64
HID = HEAD * CHANNEL

NC = 2
NS = 16
NW = NC * NS
L = 16

EPW = N_EDGES // NW
CKA = 40
NCH_A = EPW // CKA

CKC = 100
NPASS = 4
CW = HID // NPASS
NRPT = N_NODES // NS

BM = 2000

_NEG_INF = -3.0e38


def _mm_kern(a_ref, b_ref, bias_ref, o_ref):
    o_ref[...] = (
        jnp.dot(a_ref[...], b_ref[...], preferred_element_type=jnp.float32)
        + bias_ref[...]
    )


def _matmul_bias(a, b, bias):
    m, k = a.shape
    _, n = b.shape
    return pl.pallas_call(
        _mm_kern,
        grid=(m // BM,),
        in_specs=[
            pl.BlockSpec((BM, k), lambda i: (i, 0)),
            pl.BlockSpec((k, n), lambda i: (0, 0)),
            pl.BlockSpec((1, n), lambda i: (0, 0)),
        ],
        out_specs=pl.BlockSpec((BM, n), lambda i: (i, 0)),
        out_shape=jax.ShapeDtypeStruct((m, n), jnp.float32),
    )(a, b, bias.reshape(1, n))


def _mm_h4_kern(a_ref, b_ref, bias_ref, o_ref, o4_ref):
    acc = (
        jnp.dot(a_ref[...], b_ref[...], preferred_element_type=jnp.float32)
        + bias_ref[...]
    )
    o_ref[...] = acc
    for i in range(NPASS):
        o4_ref[i] = acc[:, i * CW:(i + 1) * CW]


def _matmul_bias_h4(a, b, bias):
    m, k = a.shape
    _, n = b.shape
    return pl.pallas_call(
        _mm_h4_kern,
        grid=(m // BM,),
        in_specs=[
            pl.BlockSpec((BM, k), lambda i: (i, 0)),
            pl.BlockSpec((k, n), lambda i: (0, 0)),
            pl.BlockSpec((1, n), lambda i: (0, 0)),
        ],
        out_specs=[
            pl.BlockSpec((BM, n), lambda i: (i, 0)),
            pl.BlockSpec((NPASS, BM, CW), lambda i: (0, i, 0)),
        ],
        out_shape=(
            jax.ShapeDtypeStruct((m, n), jnp.float32),
            jax.ShapeDtypeStruct((NPASS, m, CW), jnp.float32),
        ),
    )(a, b, bias.reshape(1, n))


def _fin_kern(o0_ref, o1_ref, s0_ref, s1_ref, e8_ref, h_ref, h4_ref):
    s = s0_ref[0] + s1_ref[0]
    r = jnp.where(s > 0.0, 1.0 / s, 0.0)
    rexp = jnp.dot(r, e8_ref[...], preferred_element_type=jnp.float32)
    hsum = (o0_ref[0, 0] + o1_ref[0, 0]) * rexp
    h_ref[...] = hsum
    h4_ref[0] = hsum


def _finalize(opart, spart, e8):
    f32 = jnp.float32
    return pl.pallas_call(
        _fin_kern,
        grid=(NPASS, N_NODES // BM),
        in_specs=[
            pl.BlockSpec((1, 1, BM, CW), lambda p, r: (0, p, r, 0)),
            pl.BlockSpec((1, 1, BM, CW), lambda p, r: (1, p, r, 0)),
            pl.BlockSpec((1, BM, HEAD), lambda p, r: (0, r, 0)),
            pl.BlockSpec((1, BM, HEAD), lambda p, r: (1, r, 0)),
            pl.BlockSpec((HEAD, CW), lambda p, r: (0, p)),
        ],
        out_specs=[
            pl.BlockSpec((BM, CW), lambda p, r: (r, p)),
            pl.BlockSpec((1, BM, CW), lambda p, r: (p, r, 0)),
        ],
        out_shape=(
            jax.ShapeDtypeStruct((N_NODES, HID), f32),
            jax.ShapeDtypeStruct((NPASS, N_NODES, CW), f32),
        ),
    )(opart, opart, spart, spart, e8)


def _sc_logits_body(p1_hbm, p2_hbm, src_hbm, dst_hbm, aw_hbm,
                    logit_hbm, pmax_hbm,
                    sidxA, didxA, rows1A,
                    sidxB, didxB, rows1B,
                    awv, sbuf, lbuf, mtmp,
                    s1A, s2A, s1B, s2B):
    cid = lax.axis_index("c")
    sid = lax.axis_index("s")
    w = sid * NC + cid
    wbase = w * EPW
    iot = lax.iota(jnp.int32, L)
    lane_par = lax.shift_right_logical(iot, 3)
    lane_head = lax.bitwise_and(iot, 7)
    splat15 = jnp.full((L,), 15, jnp.int32)

    pltpu.sync_copy(aw_hbm, awv)
    wvecs = [[awv[h, pl.ds(cb * L, L)] for cb in range(4)]
             for h in range(HEAD)]

    def compute_chunk(eb, rows, hmaxv):
        for h in range(HEAD):
            wv4 = wvecs[h]

            def edge_body(e, carry, h=h, wv4=wv4, rows=rows):
                base = h * CHANNEL
                acc = jnp.zeros((L,), jnp.float32)
                for cb in range(4):
                    z = rows[e, pl.ds(base + cb * L, L)]
                    t = jnp.maximum(z, 0.01 * z)
                    acc = acc + t * wv4[cb]
                sbuf[h, e, :] = plsc.cumsum(acc)
                return carry

            lax.fori_loop(0, CKA, edge_body, 0, unroll=4)

        def pair_body(j, hm):
            erow = 2 * j + lane_par
            row16 = plsc.load_gather(sbuf, [lane_head, erow, splat15])
            lbuf[pl.ds(j * L, L)] = row16
            return jnp.maximum(hm, row16)

        hmaxv = lax.fori_loop(0, CKA // 2, pair_body, hmaxv, unroll=2)
        pltpu.sync_copy(lbuf, logit_hbm.at[pl.ds(eb * HEAD, CKA * HEAD)])
        return hmaxv

    def pair_chunks(k2, hmaxv):
        ebA = wbase + (2 * k2) * CKA
        ebB = ebA + CKA
        pltpu.sync_copy(src_hbm.at[pl.ds(ebA, CKA)], sidxA)
        pltpu.sync_copy(dst_hbm.at[pl.ds(ebA, CKA)], didxA)
        cA1 = pltpu.async_copy(p1_hbm.at[sidxA], rows1A, s1A)
        pltpu.sync_copy(src_hbm.at[pl.ds(ebB, CKA)], sidxB)
        pltpu.sync_copy(dst_hbm.at[pl.ds(ebB, CKA)], didxB)
        cB1 = pltpu.async_copy(p1_hbm.at[sidxB], rows1B, s1B)
        cA1.wait()
        cA2 = pltpu.async_copy(p2_hbm.at[didxA], rows1A, s2A, add=True)
        cB1.wait()
        cB2 = pltpu.async_copy(p2_hbm.at[didxB], rows1B, s2B, add=True)
        cA2.wait()
        hmaxv = compute_chunk(ebA, rows1A, hmaxv)
        cB2.wait()
        hmaxv = compute_chunk(ebB, rows1B, hmaxv)
        return hmaxv

    hmaxv = lax.fori_loop(0, NCH_A // 2, pair_chunks,
                          jnp.full((L,), _NEG_INF, jnp.float32))

    ebT = wbase + (NCH_A - 1) * CKA
    pltpu.sync_copy(src_hbm.at[pl.ds(ebT, CKA)], sidxA)
    pltpu.sync_copy(dst_hbm.at[pl.ds(ebT, CKA)], didxA)
    pltpu.async_copy(p1_hbm.at[sidxA], rows1A, s1A).wait()
    pltpu.async_copy(p2_hbm.at[didxA], rows1A, s2A, add=True).wait()
    hmaxv = compute_chunk(ebT, rows1A, hmaxv)

    mtmp[:] = hmaxv
    other = plsc.load_gather(mtmp, [lax.bitwise_and(iot + 8, 15)])
    mtmp[:] = jnp.maximum(hmaxv, other)
    pltpu.sync_copy(mtmp, pmax_hbm.at[w])


def _sc_logits(P1, P2, src, dst, aw):
    f32 = jnp.float32
    i32 = jnp.int32
    kern = pl.kernel(
        _sc_logits_body,
        out_type=(
            jax.ShapeDtypeStruct((N_EDGES * HEAD,), f32),
            jax.ShapeDtypeStruct((NW, L), f32),
        ),
        mesh=plsc.VectorSubcoreMesh(core_axis_name="c", subcore_axis_name="s"),
        compiler_params=pltpu.CompilerParams(
            needs_layout_passes=False, use_tc_tiling_on_sc=False),
        scratch_types=[
            pltpu.VMEM((CKA,), i32),
            pltpu.VMEM((CKA,), i32),
            pltpu.VMEM((CKA, HID), f32),
            pltpu.VMEM((CKA,), i32),
            pltpu.VMEM((CKA,), i32),
            pltpu.VMEM((CKA, HID), f32),
            pltpu.VMEM((HEAD, CHANNEL), f32),
            pltpu.VMEM((HEAD, CKA, L), f32),
            pltpu.VMEM((CKA * HEAD,), f32),
            pltpu.VMEM((L,), f32),
            pltpu.SemaphoreType.DMA,
            pltpu.SemaphoreType.DMA,
            pltpu.SemaphoreType.DMA,
            pltpu.SemaphoreType.DMA,
        ],
    )
    return kern(P1, P2, src, dst, aw)


def _sc_agg_body(logit_hbm, pmax_hbm, h4_hbm, src2_hbm, dst2_hbm,
                 zzm_hbm, zzs_hbm,
                 opart_hbm, spart_hbm,
                 pmv, sidxA, didxA, lflatA, hrowsA,
                 sidxB, didxB, lflatB, hrowsB,
                 prows, acc_shared, s_shared,
                 gsemA, gsemB, ssemA, ssemB):
    cid = lax.axis_index("c")
    sid = lax.axis_index("s")
    w = sid * NC + cid
    wbase = w * EPW
    iot = lax.iota(jnp.int32, L)
    lane_par = lax.shift_right_logical(iot, 3)
    lane_head = lax.bitwise_and(iot, 7)

    pltpu.sync_copy(pmax_hbm, pmv)

    def mred(k, m):
        return jnp.maximum(m, pmv[k, :])
    gmax16 = lax.fori_loop(0, NW, mred, jnp.full((L,), _NEG_INF, jnp.float32))

    nch = EPW // CKC

    for pp in range(NPASS):
        pltpu.sync_copy(zzm_hbm, acc_shared.at[pl.ds(sid * NRPT, NRPT)])
        if pp == 0:
            pltpu.sync_copy(zzs_hbm, s_shared.at[pl.ds(sid * NRPT, NRPT)])
        plsc.subcore_barrier()

        cv0 = jnp.full((L,), 2 * pp, jnp.int32)
        cv1 = jnp.full((L,), 2 * pp + 1, jnp.int32)

        def half_chunk(eb, sidx, didx, lflat, hrows, gdesc, ssem,
                       pp=pp, cv0=cv0, cv1=cv1):
            def gbody(k, c2):
                v = lflat[pl.ds(k * L, L)]
                pv = jnp.exp(v - gmax16)
                plsc.store_scatter(prows, [2 * k + lane_par, lane_head], pv)
                return c2
            lax.fori_loop(0, CKC * HEAD // L, gbody, 0, unroll=4)

            gdesc.wait()

            if pp == 0:
                pltpu.sync_copy(prows, s_shared.at[didx.at[0]], add=True)

            def ebody(e, c2):
                erow = jnp.full((L,), 0, jnp.int32) + e
                a0 = plsc.load_gather(prows, [erow, cv0])
                a1 = plsc.load_gather(prows, [erow, cv1])
                for cb in range(8):
                    a = a0 if cb < 4 else a1
                    hrows[e, pl.ds(cb * L, L)] = (
                        hrows[e, pl.ds(cb * L, L)] * a)
                return c2
            lax.fori_loop(0, CKC, ebody, 0, unroll=2)

            return pltpu.async_copy(hrows, acc_shared.at[didx.at[0]], ssem,
                                    add=True)

        def pair_body(k2, c, pp=pp, cv0=cv0, cv1=cv1):
            ebA = wbase + (2 * k2) * CKC
            ebB = ebA + CKC
            pltpu.sync_copy(src2_hbm.at[pl.ds(ebA // CKC, 1)], sidxA)
            pltpu.sync_copy(dst2_hbm.at[pl.ds(ebA // CKC, 1)], didxA)
            gA = pltpu.async_copy(h4_hbm.at[pp].at[sidxA.at[0]], hrowsA, gsemA)
            pltpu.sync_copy(src2_hbm.at[pl.ds(ebB // CKC, 1)], sidxB)
            pltpu.sync_copy(dst2_hbm.at[pl.ds(ebB // CKC, 1)], didxB)
            gB = pltpu.async_copy(h4_hbm.at[pp].at[sidxB.at[0]], hrowsB, gsemB)
            pltpu.sync_copy(logit_hbm.at[pl.ds(ebA * HEAD, CKC * HEAD)],
                            lflatA)
            pltpu.sync_copy(logit_hbm.at[pl.ds(ebB * HEAD, CKC * HEAD)],
                            lflatB)
            sA = half_chunk(ebA, sidxA, didxA, lflatA, hrowsA, gA, ssemA)
            sB = half_chunk(ebB, sidxB, didxB, lflatB, hrowsB, gB, ssemB)
            sA.wait()
            sB.wait()
            return c

        lax.fori_loop(0, nch // 2, pair_body, 0)
        plsc.subcore_barrier()
        if pp == 0:
            pltpu.sync_copy(s_shared.at[pl.ds(sid * NRPT, NRPT)],
                            spart_hbm.at[cid].at[pl.ds(sid * NRPT, NRPT)])
        pltpu.sync_copy(acc_shared.at[pl.ds(sid * NRPT, NRPT)],
                        opart_hbm.at[cid].at[pp].at[pl.ds(sid * NRPT, NRPT)])


def _sc_agg(logit_flat, pmax, h4, src2, dst2, zzm, zzs):
    f32 = jnp.float32
    i32 = jnp.int32
    kern = pl.kernel(
        _sc_agg_body,
        out_type=(
            jax.ShapeDtypeStruct((NC, NPASS, N_NODES, CW), f32),
            jax.ShapeDtypeStruct((NC, N_NODES, HEAD), f32),
        ),
        mesh=plsc.VectorSubcoreMesh(core_axis_name="c", subcore_axis_name="s"),
        compiler_params=pltpu.CompilerParams(
            needs_layout_passes=False, use_tc_tiling_on_sc=False),
        scratch_types=[
            pltpu.VMEM((NW, L), f32),
            pltpu.VMEM((1, CKC), i32),
            pltpu.VMEM((1, CKC), i32),
            pltpu.VMEM((CKC * HEAD,), f32),
            pltpu.VMEM((CKC, CW), f32),
            pltpu.VMEM((1, CKC), i32),
            pltpu.VMEM((1, CKC), i32),
            pltpu.VMEM((CKC * HEAD,), f32),
            pltpu.VMEM((CKC, CW), f32),
            pltpu.VMEM((CKC, HEAD), f32),
            pltpu.VMEM_SHARED((N_NODES, CW), f32),
            pltpu.VMEM_SHARED((N_NODES, HEAD), f32),
            pltpu.SemaphoreType.DMA,
            pltpu.SemaphoreType.DMA,
            pltpu.SemaphoreType.DMA,
            pltpu.SemaphoreType.DMA,
        ],
    )
    return kern(logit_flat, pmax, h4, src2, dst2, zzm, zzs)


def kernel(x, edge_attr, edge_index, Wn, bn, We, be, Wa, ba, attn_w):
    f32 = jnp.float32
    src = edge_index[0].astype(jnp.int32)
    dst = edge_index[1].astype(jnp.int32)
    src2 = src.reshape(N_EDGES // CKC, CKC)
    dst2 = dst.reshape(N_EDGES // CKC, CKC)
    zzm = jnp.zeros((NRPT, CW), f32)
    zzs = jnp.zeros((NRPT, HEAD), f32)
    e8 = jnp.repeat(jnp.eye(HEAD, dtype=f32), CHANNEL, axis=1)

    xp = jnp.pad(x, ((0, 0), (0, 10)))
    Wnp = jnp.pad(Wn, ((0, 10), (0, 0)))
    h, h4 = _matmul_bias_h4(xp, Wnp, bn)

    W2 = jnp.concatenate([Wa[:HID], Wa[HID:]], axis=1)
    b2 = jnp.concatenate([ba, jnp.zeros((HID,), f32)])
    aw = attn_w.reshape(HEAD, CHANNEL)

    for _ in range(2):
        P = _matmul_bias(h, W2, b2)
        logit_flat, pmax = _sc_logits(P[:, :HID], P[:, HID:], src, dst, aw)
        opart, spart = _sc_agg(logit_flat, pmax, h4, src2, dst2, zzm, zzs)
        h, h4 = _finalize(opart, spart, e8)
    return h


# speedup vs baseline: 1.0248x; 1.0248x over previous
import functools
import jax
import jax.numpy as jnp
from jax import lax
from jax.experimental import pallas as pl
from jax.experimental.pallas import tpu as pltpu
from jax.experimental.pallas import tpu_sc as plsc

N_NODES = 10000
N_EDGES = 160000
HEAD = 8
CHANNEL = 64
HID = HEAD * CHANNEL

NC = 2
NS = 16
NW = NC * NS
L = 16

EPW = N_EDGES // NW
CKA = 40
NCH_A = EPW // CKA

CKC = 100
NPASS = 4
CW = HID // NPASS
NRPT = N_NODES // NS

BM = 2000

_NEG_INF = -3.0e38


def _mm_kern(a_ref, b_ref, bias_ref, o_ref):
    o_ref[...] = (
        jnp.dot(a_ref[...], b_ref[...], preferred_element_type=jnp.float32)
        + bias_ref[...]
    )


def _matmul_bias(a, b, bias):
    m, k = a.shape
    _, n = b.shape
    return pl.pallas_call(
        _mm_kern,
        grid=(m // BM,),
        in_specs=[
            pl.BlockSpec((BM, k), lambda i: (i, 0)),
            pl.BlockSpec((k, n), lambda i: (0, 0)),
            pl.BlockSpec((1, n), lambda i: (0, 0)),
        ],
        out_specs=pl.BlockSpec((BM, n), lambda i: (i, 0)),
        out_shape=jax.ShapeDtypeStruct((m, n), jnp.float32),
    )(a, b, bias.reshape(1, n))


def _mm_h4_kern(a_ref, b_ref, bias_ref, o_ref, o4_ref):
    acc = (
        jnp.dot(a_ref[...], b_ref[...], preferred_element_type=jnp.float32)
        + bias_ref[...]
    )
    o_ref[...] = acc
    for i in range(NPASS):
        o4_ref[i] = acc[:, i * CW:(i + 1) * CW]


def _matmul_bias_h4(a, b, bias):
    m, k = a.shape
    _, n = b.shape
    return pl.pallas_call(
        _mm_h4_kern,
        grid=(m // BM,),
        in_specs=[
            pl.BlockSpec((BM, k), lambda i: (i, 0)),
            pl.BlockSpec((k, n), lambda i: (0, 0)),
            pl.BlockSpec((1, n), lambda i: (0, 0)),
        ],
        out_specs=[
            pl.BlockSpec((BM, n), lambda i: (i, 0)),
            pl.BlockSpec((NPASS, BM, CW), lambda i: (0, i, 0)),
        ],
        out_shape=(
            jax.ShapeDtypeStruct((m, n), jnp.float32),
            jax.ShapeDtypeStruct((NPASS, m, CW), jnp.float32),
        ),
    )(a, b, bias.reshape(1, n))


def _fin_kern(o0_ref, o1_ref, s0_ref, s1_ref, e8_ref, h_ref, h4_ref):
    s = s0_ref[0] + s1_ref[0]
    r = jnp.where(s > 0.0, 1.0 / s, 0.0)
    rexp = jnp.dot(r, e8_ref[...], preferred_element_type=jnp.float32)
    hsum = (o0_ref[0, 0] + o1_ref[0, 0]) * rexp
    h_ref[...] = hsum
    h4_ref[0] = hsum


def _finalize(opart, spart, e8):
    f32 = jnp.float32
    return pl.pallas_call(
        _fin_kern,
        grid=(NPASS, N_NODES // BM),
        in_specs=[
            pl.BlockSpec((1, 1, BM, CW), lambda p, r: (0, p, r, 0)),
            pl.BlockSpec((1, 1, BM, CW), lambda p, r: (1, p, r, 0)),
            pl.BlockSpec((1, BM, HEAD), lambda p, r: (0, r, 0)),
            pl.BlockSpec((1, BM, HEAD), lambda p, r: (1, r, 0)),
            pl.BlockSpec((HEAD, CW), lambda p, r: (0, p)),
        ],
        out_specs=[
            pl.BlockSpec((BM, CW), lambda p, r: (r, p)),
            pl.BlockSpec((1, BM, CW), lambda p, r: (p, r, 0)),
        ],
        out_shape=(
            jax.ShapeDtypeStruct((N_NODES, HID), f32),
            jax.ShapeDtypeStruct((NPASS, N_NODES, CW), f32),
        ),
    )(opart, opart, spart, spart, e8)


def _sc_logits_body(p1_hbm, p2_hbm, src_hbm, dst_hbm, aw_hbm,
                    logit_hbm, pmax_hbm,
                    sidxA, didxA, rows1A,
                    sidxB, didxB, rows1B,
                    awv, sbuf, lbuf, mtmp,
                    s1A, s2A, s1B, s2B):
    cid = lax.axis_index("c")
    sid = lax.axis_index("s")
    w = sid * NC + cid
    wbase = w * EPW
    iot = lax.iota(jnp.int32, L)
    lane_par = lax.shift_right_logical(iot, 3)
    lane_head = lax.bitwise_and(iot, 7)
    splat15 = jnp.full((L,), 15, jnp.int32)

    pltpu.sync_copy(aw_hbm, awv)
    wvecs = [[awv[h, pl.ds(cb * L, L)] for cb in range(4)]
             for h in range(HEAD)]

    def compute_chunk(eb, rows, hmaxv):
        for h in range(HEAD):
            wv4 = wvecs[h]

            def edge_body(e, carry, h=h, wv4=wv4, rows=rows):
                base = h * CHANNEL
                acc = jnp.zeros((L,), jnp.float32)
                for cb in range(4):
                    z = rows[e, pl.ds(base + cb * L, L)]
                    t = jnp.maximum(z, 0.01 * z)
                    acc = acc + t * wv4[cb]
                sbuf[h, e, :] = plsc.cumsum(acc)
                return carry

            lax.fori_loop(0, CKA, edge_body, 0, unroll=2)

        def pair_body(j, hm):
            erow = 2 * j + lane_par
            row16 = plsc.load_gather(sbuf, [lane_head, erow, splat15])
            lbuf[pl.ds(j * L, L)] = row16
            return jnp.maximum(hm, row16)

        hmaxv = lax.fori_loop(0, CKA // 2, pair_body, hmaxv, unroll=2)
        pltpu.sync_copy(lbuf, logit_hbm.at[pl.ds(eb * HEAD, CKA * HEAD)])
        return hmaxv

    def pair_chunks(k2, hmaxv):
        ebA = wbase + (2 * k2) * CKA
        ebB = ebA + CKA
        pltpu.sync_copy(src_hbm.at[pl.ds(ebA, CKA)], sidxA)
        pltpu.sync_copy(dst_hbm.at[pl.ds(ebA, CKA)], didxA)
        cA1 = pltpu.async_copy(p1_hbm.at[sidxA], rows1A, s1A)
        pltpu.sync_copy(src_hbm.at[pl.ds(ebB, CKA)], sidxB)
        pltpu.sync_copy(dst_hbm.at[pl.ds(ebB, CKA)], didxB)
        cB1 = pltpu.async_copy(p1_hbm.at[sidxB], rows1B, s1B)
        cA1.wait()
        cA2 = pltpu.async_copy(p2_hbm.at[didxA], rows1A, s2A, add=True)
        cB1.wait()
        cB2 = pltpu.async_copy(p2_hbm.at[didxB], rows1B, s2B, add=True)
        cA2.wait()
        hmaxv = compute_chunk(ebA, rows1A, hmaxv)
        cB2.wait()
        hmaxv = compute_chunk(ebB, rows1B, hmaxv)
        return hmaxv

    hmaxv = lax.fori_loop(0, NCH_A // 2, pair_chunks,
                          jnp.full((L,), _NEG_INF, jnp.float32))

    ebT = wbase + (NCH_A - 1) * CKA
    pltpu.sync_copy(src_hbm.at[pl.ds(ebT, CKA)], sidxA)
    pltpu.sync_copy(dst_hbm.at[pl.ds(ebT, CKA)], didxA)
    pltpu.async_copy(p1_hbm.at[sidxA], rows1A, s1A).wait()
    pltpu.async_copy(p2_hbm.at[didxA], rows1A, s2A, add=True).wait()
    hmaxv = compute_chunk(ebT, rows1A, hmaxv)

    mtmp[:] = hmaxv
    other = plsc.load_gather(mtmp, [lax.bitwise_and(iot + 8, 15)])
    mtmp[:] = jnp.maximum(hmaxv, other)
    pltpu.sync_copy(mtmp, pmax_hbm.at[w])


def _sc_logits(P1, P2, src, dst, aw):
    f32 = jnp.float32
    i32 = jnp.int32
    kern = pl.kernel(
        _sc_logits_body,
        out_type=(
            jax.ShapeDtypeStruct((N_EDGES * HEAD,), f32),
            jax.ShapeDtypeStruct((NW, L), f32),
        ),
        mesh=plsc.VectorSubcoreMesh(core_axis_name="c", subcore_axis_name="s"),
        compiler_params=pltpu.CompilerParams(
            needs_layout_passes=False, use_tc_tiling_on_sc=False),
        scratch_types=[
            pltpu.VMEM((CKA,), i32),
            pltpu.VMEM((CKA,), i32),
            pltpu.VMEM((CKA, HID), f32),
            pltpu.VMEM((CKA,), i32),
            pltpu.VMEM((CKA,), i32),
            pltpu.VMEM((CKA, HID), f32),
            pltpu.VMEM((HEAD, CHANNEL), f32),
            pltpu.VMEM((HEAD, CKA, L), f32),
            pltpu.VMEM((CKA * HEAD,), f32),
            pltpu.VMEM((L,), f32),
            pltpu.SemaphoreType.DMA,
            pltpu.SemaphoreType.DMA,
            pltpu.SemaphoreType.DMA,
            pltpu.SemaphoreType.DMA,
        ],
    )
    return kern(P1, P2, src, dst, aw)


def _sc_agg_body(logit_hbm, pmax_hbm, h4_hbm, src2_hbm, dst2_hbm,
                 zzm_hbm, zzs_hbm,
                 opart_hbm, spart_hbm,
                 pmv, sidxA, didxA, lflatA, hrowsA,
                 sidxB, didxB, lflatB, hrowsB,
                 prows, acc_shared, s_shared,
                 gsemA, gsemB, ssemA, ssemB):
    cid = lax.axis_index("c")
    sid = lax.axis_index("s")
    w = sid * NC + cid
    wbase = w * EPW
    iot = lax.iota(jnp.int32, L)
    lane_par = lax.shift_right_logical(iot, 3)
    lane_head = lax.bitwise_and(iot, 7)

    pltpu.sync_copy(pmax_hbm, pmv)

    def mred(k, m):
        return jnp.maximum(m, pmv[k, :])
    gmax16 = lax.fori_loop(0, NW, mred, jnp.full((L,), _NEG_INF, jnp.float32))

    nch = EPW // CKC

    for pp in range(NPASS):
        pltpu.sync_copy(zzm_hbm, acc_shared.at[pl.ds(sid * NRPT, NRPT)])
        if pp == 0:
            pltpu.sync_copy(zzs_hbm, s_shared.at[pl.ds(sid * NRPT, NRPT)])
        plsc.subcore_barrier()

        cv0 = jnp.full((L,), 2 * pp, jnp.int32)
        cv1 = jnp.full((L,), 2 * pp + 1, jnp.int32)

        def half_chunk(eb, sidx, didx, lflat, hrows, gdesc, ssem,
                       pp=pp, cv0=cv0, cv1=cv1):
            def gbody(k, c2):
                v = lflat[pl.ds(k * L, L)]
                pv = jnp.exp(v - gmax16)
                plsc.store_scatter(prows, [2 * k + lane_par, lane_head], pv)
                return c2
            lax.fori_loop(0, CKC * HEAD // L, gbody, 0, unroll=4)

            gdesc.wait()

            if pp == 0:
                pltpu.sync_copy(prows, s_shared.at[didx.at[0]], add=True)

            def ebody(e, c2):
                erow = jnp.full((L,), 0, jnp.int32) + e
                a0 = plsc.load_gather(prows, [erow, cv0])
                a1 = plsc.load_gather(prows, [erow, cv1])
                for cb in range(8):
                    a = a0 if cb < 4 else a1
                    hrows[e, pl.ds(cb * L, L)] = (
                        hrows[e, pl.ds(cb * L, L)] * a)
                return c2
            lax.fori_loop(0, CKC, ebody, 0, unroll=2)

            return pltpu.async_copy(hrows, acc_shared.at[didx.at[0]], ssem,
                                    add=True)

        def pair_body(k2, c, pp=pp, cv0=cv0, cv1=cv1):
            ebA = wbase + (2 * k2) * CKC
            ebB = ebA + CKC
            pltpu.sync_copy(src2_hbm.at[pl.ds(ebA // CKC, 1)], sidxA)
            pltpu.sync_copy(dst2_hbm.at[pl.ds(ebA // CKC, 1)], didxA)
            gA = pltpu.async_copy(h4_hbm.at[pp].at[sidxA.at[0]], hrowsA, gsemA)
            pltpu.sync_copy(src2_hbm.at[pl.ds(ebB // CKC, 1)], sidxB)
            pltpu.sync_copy(dst2_hbm.at[pl.ds(ebB // CKC, 1)], didxB)
            gB = pltpu.async_copy(h4_hbm.at[pp].at[sidxB.at[0]], hrowsB, gsemB)
            pltpu.sync_copy(logit_hbm.at[pl.ds(ebA * HEAD, CKC * HEAD)],
                            lflatA)
            pltpu.sync_copy(logit_hbm.at[pl.ds(ebB * HEAD, CKC * HEAD)],
                            lflatB)
            sA = half_chunk(ebA, sidxA, didxA, lflatA, hrowsA, gA, ssemA)
            sB = half_chunk(ebB, sidxB, didxB, lflatB, hrowsB, gB, ssemB)
            sA.wait()
            sB.wait()
            return c

        lax.fori_loop(0, nch // 2, pair_body, 0)
        plsc.subcore_barrier()
        if pp == 0:
            pltpu.sync_copy(s_shared.at[pl.ds(sid * NRPT, NRPT)],
                            spart_hbm.at[cid].at[pl.ds(sid * NRPT, NRPT)])
        pltpu.sync_copy(acc_shared.at[pl.ds(sid * NRPT, NRPT)],
                        opart_hbm.at[cid].at[pp].at[pl.ds(sid * NRPT, NRPT)])


def _sc_agg(logit_flat, pmax, h4, src2, dst2, zzm, zzs):
    f32 = jnp.float32
    i32 = jnp.int32
    kern = pl.kernel(
        _sc_agg_body,
        out_type=(
            jax.ShapeDtypeStruct((NC, NPASS, N_NODES, CW), f32),
            jax.ShapeDtypeStruct((NC, N_NODES, HEAD), f32),
        ),
        mesh=plsc.VectorSubcoreMesh(core_axis_name="c", subcore_axis_name="s"),
        compiler_params=pltpu.CompilerParams(
            needs_layout_passes=False, use_tc_tiling_on_sc=False),
        scratch_types=[
            pltpu.VMEM((NW, L), f32),
            pltpu.VMEM((1, CKC), i32),
            pltpu.VMEM((1, CKC), i32),
            pltpu.VMEM((CKC * HEAD,), f32),
            pltpu.VMEM((CKC, CW), f32),
            pltpu.VMEM((1, CKC), i32),
            pltpu.VMEM((1, CKC), i32),
            pltpu.VMEM((CKC * HEAD,), f32),
            pltpu.VMEM((CKC, CW), f32),
            pltpu.VMEM((CKC, HEAD), f32),
            pltpu.VMEM_SHARED((N_NODES, CW), f32),
            pltpu.VMEM_SHARED((N_NODES, HEAD), f32),
            pltpu.SemaphoreType.DMA,
            pltpu.SemaphoreType.DMA,
            pltpu.SemaphoreType.DMA,
            pltpu.SemaphoreType.DMA,
        ],
    )
    return kern(logit_flat, pmax, h4, src2, dst2, zzm, zzs)


def kernel(x, edge_attr, edge_index, Wn, bn, We, be, Wa, ba, attn_w):
    f32 = jnp.float32
    src = edge_index[0].astype(jnp.int32)
    dst = edge_index[1].astype(jnp.int32)
    src2 = src.reshape(N_EDGES // CKC, CKC)
    dst2 = dst.reshape(N_EDGES // CKC, CKC)
    zzm = jnp.zeros((NRPT, CW), f32)
    zzs = jnp.zeros((NRPT, HEAD), f32)
    e8 = jnp.repeat(jnp.eye(HEAD, dtype=f32), CHANNEL, axis=1)

    xp = jnp.pad(x, ((0, 0), (0, 10)))
    Wnp = jnp.pad(Wn, ((0, 10), (0, 0)))
    h, h4 = _matmul_bias_h4(xp, Wnp, bn)

    W2 = jnp.concatenate([Wa[:HID], Wa[HID:]], axis=1)
    b2 = jnp.concatenate([ba, jnp.zeros((HID,), f32)])
    aw = attn_w.reshape(HEAD, CHANNEL)

    for _ in range(2):
        P = _matmul_bias(h, W2, b2)
        logit_flat, pmax = _sc_logits(P[:, :HID], P[:, HID:], src, dst, aw)
        opart, spart = _sc_agg(logit_flat, pmax, h4, src2, dst2, zzm, zzs)
        h, h4 = _finalize(opart, spart, e8)
    return h
